# Initial kernel scaffold; baseline (speedup 1.0000x reference)
import jax
import jax.numpy as jnp
from jax.experimental import pallas as pl


def kernel(events):
    raise NotImplementedError("write your pallas kernel here")


# speedup vs baseline: 5.7441x; 5.7441x over previous
import functools

import jax
import jax.numpy as jnp
import numpy as np
from jax import lax
from jax.experimental import pallas as pl
from jax.experimental.pallas import tpu as pltpu
from jax.experimental.pallas import tpu_sc as plsc

C, H, W = 9, 260, 346
N = 4_000_000
NV = C * H * W
GRID_PAD = 811_008
SENT = NV
NC, NS = 2, 16
NW = NC * NS
ET = N // NW
EV_CHUNK = 8192
FULL_CHUNKS = 15
TAIL = ET - FULL_CHUNKS * EV_CHUNK
TAIL_ROWS = (TAIL + 127) // 128
PER_TILE_GRID = GRID_PAD // NS

_WH = np.float32(W * H)
_Wf = np.float32(W)
_CJ = [np.float32(j / C) for j in range(1, C)]


def _tmax_body(ev_ref, out_ref):
    i = pl.program_id(0)
    blk = ev_ref[...]
    lanes = lax.broadcasted_iota(jnp.int32, blk.shape, 1)
    m = jnp.max(jnp.where(lanes % 4 == 2, blk, -jnp.inf))

    @pl.when(i == 0)
    def _():
        out_ref[0, 0] = m

    @pl.when(i != 0)
    def _():
        out_ref[0, 0] = jnp.maximum(out_ref[0, 0], m)


def _merge_body(a_ref, o_ref):
    o_ref[...] = a_ref[0] + a_ref[1]


def _sc_body(ev_hbm, tmax_hbm, out_hbm, grid_sh, ev_v, idx_v, ones_v,
             tmax_v, sem0, sem1):
    c_ax = lax.axis_index("c")
    s_ax = lax.axis_index("s")
    wid = c_ax * NS + s_ax
    ev_base = wid * (ET * 4)
    lane = lax.iota(jnp.int32, 16)
    lane4 = lane * 4

    CF = EV_CHUNK * 4

    def full_copy(cc, par):
        src = ev_hbm.at[pl.ds(ev_base + cc * CF, CF)]
        dst = ev_v.at[pl.ds(par * CF, CF)]
        return src, dst, (sem0 if par == 0 else sem1)

    def tail_copy():
        src = ev_hbm.at[pl.ds(ev_base + FULL_CHUNKS * CF, TAIL * 4)]
        dst = ev_v.at[pl.ds((FULL_CHUNKS % 2) * CF, TAIL * 4)]
        return src, dst, (sem0 if FULL_CHUNKS % 2 == 0 else sem1)

    def start_full(cc):
        for par in (0, 1):
            @pl.when(lax.rem(cc, 2) == par)
            def _():
                pltpu.async_copy(*full_copy(cc, par))

    def wait_full(cc):
        for par in (0, 1):
            @pl.when(lax.rem(cc, 2) == par)
            def _():
                pltpu.make_async_copy(*full_copy(cc, par)).wait()

    pltpu.async_copy(*full_copy(0, 0))

    zeros16 = jnp.zeros((16,), jnp.float32)

    def _zbody(i, _):
        ev_v[pl.ds(EV_CHUNK * 4 + i * 16, 16)] = zeros16
        return ()

    lax.fori_loop(0, EV_CHUNK * 4 // 16, _zbody, ())
    zoff = s_ax * PER_TILE_GRID
    pltpu.sync_copy(ev_v.at[pl.ds(EV_CHUNK * 4, EV_CHUNK * 4)],
                    grid_sh.at[pl.ds(zoff, EV_CHUNK * 4)])
    rest = PER_TILE_GRID - EV_CHUNK * 4
    pltpu.sync_copy(ev_v.at[pl.ds(EV_CHUNK * 4, rest)],
                    grid_sh.at[pl.ds(zoff + EV_CHUNK * 4, rest)])

    for m in range(8):
        ones_v[pl.ds(m * 16, 16)] = jnp.ones((16,), jnp.float32)
    pltpu.sync_copy(tmax_hbm, tmax_v)
    tmaxvec = tmax_v[...]

    plsc.subcore_barrier()

    def compute16(fo):
        ids = fo + lane4
        xv = plsc.load_gather(ev_v, [ids])
        yv = plsc.load_gather(ev_v, [ids + 1])
        tv = plsc.load_gather(ev_v, [ids + 2])
        tn = tv / tmaxvec
        base = jnp.where(tn > _CJ[0], _WH, np.float32(0.0))
        for j in range(1, 8):
            base = base + jnp.where(tn > _CJ[j], _WH, np.float32(0.0))
        s = (xv + _Wf * yv) + base
        idx = s.astype(jnp.int32)
        valid = jnp.logical_and(tn > np.float32(0.0), idx < NV)
        return jnp.where(valid, idx, SENT)

    def chunk_compute(buf_off, nrows):
        def qbody(q, _):
            fo = buf_off + q * 512
            for m in range(8):
                idx_v[q, pl.ds(m * 16, 16)] = compute16(fo + m * 64)
            return ()

        lax.fori_loop(0, nrows, qbody, ())

    def tail_compute(buf_off):
        def qbody(q, _):
            fo = buf_off + q * 512
            eid0 = q * 128
            for m in range(8):
                vec = compute16(fo + m * 64)
                eid = eid0 + m * 16 + lane
                idx_v[q, pl.ds(m * 16, 16)] = jnp.where(eid < TAIL, vec, SENT)
            return ()

        lax.fori_loop(0, TAIL_ROWS, qbody, ())

    def chunk_scatter(nrows):
        def jbody(j, _):
            pltpu.sync_copy(ones_v, grid_sh.at[idx_v.at[j]], add=True)
            return ()

        lax.fori_loop(0, nrows, jbody, ())

    def cbody(c, _):
        @pl.when(c < FULL_CHUNKS - 1)
        def _():
            start_full(c + 1)

        @pl.when(c == FULL_CHUNKS - 1)
        def _():
            pltpu.async_copy(*tail_copy())

        wait_full(c)
        chunk_compute(lax.rem(c, 2) * CF, EV_CHUNK // 128)
        chunk_scatter(EV_CHUNK // 128)
        return ()

    lax.fori_loop(0, FULL_CHUNKS, cbody, ())

    pltpu.make_async_copy(*tail_copy()).wait()
    tail_compute((FULL_CHUNKS % 2) * CF)
    chunk_scatter(TAIL_ROWS)

    plsc.subcore_barrier()
    ooff = s_ax * PER_TILE_GRID
    pltpu.sync_copy(grid_sh.at[pl.ds(ooff, PER_TILE_GRID)],
                    out_hbm.at[c_ax, pl.ds(ooff, PER_TILE_GRID)])


def _make_sc_call():
    mesh = plsc.VectorSubcoreMesh(core_axis_name="c", subcore_axis_name="s",
                                  num_cores=NC, num_subcores=NS)
    return pl.kernel(
        _sc_body,
        out_type=jax.ShapeDtypeStruct((NC, GRID_PAD), jnp.float32),
        mesh=mesh,
        compiler_params=pltpu.CompilerParams(needs_layout_passes=False),
        scratch_types=[
            pltpu.VMEM_SHARED((GRID_PAD,), jnp.float32),
            pltpu.VMEM((2 * EV_CHUNK * 4,), jnp.float32),
            pltpu.VMEM((64, 128), jnp.int32),
            pltpu.VMEM((128,), jnp.float32),
            pltpu.VMEM((16,), jnp.float32),
            pltpu.SemaphoreType.DMA,
            pltpu.SemaphoreType.DMA,
        ],
    )


@jax.jit
def kernel(events):
    ev2 = events.reshape(25_000, 640)
    tmax = pl.pallas_call(
        _tmax_body,
        grid=(25,),
        in_specs=[pl.BlockSpec((1000, 640), lambda i: (i, 0))],
        out_specs=pl.BlockSpec(memory_space=pltpu.SMEM),
        out_shape=jax.ShapeDtypeStruct((1, 1), jnp.float32),
    )(ev2)
    tmax16 = jnp.broadcast_to(tmax.reshape(1), (16,))

    partials = _make_sc_call()(events.reshape(N * 4), tmax16)

    p3 = partials.reshape(NC, GRID_PAD // 128, 128)
    merged = pl.pallas_call(
        _merge_body,
        grid=(8,),
        in_specs=[pl.BlockSpec((NC, GRID_PAD // 128 // 8, 128),
                               lambda i: (0, i, 0))],
        out_specs=pl.BlockSpec((GRID_PAD // 128 // 8, 128), lambda i: (i, 0)),
        out_shape=jax.ShapeDtypeStruct((GRID_PAD // 128, 128), jnp.float32),
    )(p3)
    return merged.reshape(-1)[:NV].reshape(1, C, H, W)
